# baseline (speedup 1.0000x reference)
import functools

import jax
import jax.numpy as jnp
from jax import lax
from jax.experimental import pallas as pl
from jax.experimental.pallas import tpu as pltpu
from jax.experimental.pallas import tpu_sc as plsc

VOCAB = 10
HIDDEN = 4096
PRE = 16
BATCH = 4
SEQ = 8192
TOT = PRE + SEQ

NC = 2
NS = 16
NW = NC * NS
ROWS_PER_W = BATCH * SEQ // NW
L = 16
NCH = ROWS_PER_W // L
WPB = NW // BATCH
PQ = PRE // WPB


def _sc_embed(ids3, wef, pef):
    mesh = plsc.VectorSubcoreMesh(core_axis_name="c", subcore_axis_name="s")

    @functools.partial(
        pl.kernel,
        mesh=mesh,
        compiler_params=pltpu.CompilerParams(needs_layout_passes=False),
        out_type=jax.ShapeDtypeStruct((BATCH * TOT * HIDDEN,), jnp.float32),
        scratch_types=[
            pltpu.VMEM((NCH, L), jnp.int32),
            pltpu.VMEM((VOCAB * HIDDEN,), jnp.float32),
            pltpu.VMEM((PQ * HIDDEN,), jnp.float32),
            pltpu.SemaphoreType.DMA,
            pltpu.SemaphoreType.DMA,
        ],
    )
    def k(ids_hbm, we_hbm, pe_hbm, out_hbm, idx_v, tab_v, pe_v, sem, sp):
        wid = lax.axis_index("s") * NC + lax.axis_index("c")
        b = wid // WPB
        q = wid % WPB
        s0 = q * ROWS_PER_W
        pltpu.sync_copy(ids_hbm.at[wid], idx_v)
        pltpu.sync_copy(we_hbm, tab_v)
        p_dst = (b * TOT + q * PQ) * HIDDEN
        pltpu.sync_copy(pe_hbm.at[pl.ds(q * PQ * HIDDEN, PQ * HIDDEN)], pe_v)
        pltpu.make_async_copy(
            pe_v, out_hbm.at[pl.ds(p_dst, PQ * HIDDEN)], sp
        ).start()

        base = (b * TOT + PRE + s0) * HIDDEN

        def body(c, carry):
            vec = idx_v[c]
            r0 = base + c * (L * HIDDEN)
            for j in range(L):
                sid = vec[j]
                pltpu.make_async_copy(
                    tab_v.at[pl.ds(sid * HIDDEN, HIDDEN)],
                    out_hbm.at[pl.ds(r0 + j * HIDDEN, HIDDEN)],
                    sem,
                ).start()
            return carry

        lax.fori_loop(0, NCH, body, 0)
        pltpu.make_async_copy(
            out_hbm.at[pl.ds(base, ROWS_PER_W * HIDDEN)],
            out_hbm.at[pl.ds(base, ROWS_PER_W * HIDDEN)],
            sem,
        ).wait()
        pltpu.make_async_copy(
            pe_v, out_hbm.at[pl.ds(p_dst, PQ * HIDDEN)], sp
        ).wait()

    return k(ids3, wef, pef)


@jax.jit
def kernel(input_ids, word_embeddings, prompt_embeddings):
    ids3 = input_ids.astype(jnp.int32).reshape(NW, NCH, L)
    wef = word_embeddings.reshape(VOCAB * HIDDEN)
    pef = prompt_embeddings.reshape(PRE * HIDDEN)
    out = _sc_embed(ids3, wef, pef)
    return out.reshape(BATCH, TOT, HIDDEN)


# speedup vs baseline: 3.5175x; 3.5175x over previous
import functools

import jax
import jax.numpy as jnp
from jax import lax
from jax.experimental import pallas as pl
from jax.experimental.pallas import tpu as pltpu
from jax.experimental.pallas import tpu_sc as plsc

VOCAB = 10
HIDDEN = 4096
PRE = 16
BATCH = 4
SEQ = 8192

NC = 2
NS = 16
NW = NC * NS
ROWS_PER_W = BATCH * SEQ // NW
L = 16
NCH = ROWS_PER_W // L
WPB = NW // BATCH


def _sc_embed(ids3, word_embeddings, prompt_embeddings):
    mesh = plsc.VectorSubcoreMesh(core_axis_name="c", subcore_axis_name="s")

    @functools.partial(
        pl.kernel,
        mesh=mesh,
        compiler_params=pltpu.CompilerParams(needs_layout_passes=False),
        out_type=jax.ShapeDtypeStruct((BATCH, PRE + SEQ, HIDDEN), jnp.float32),
        scratch_types=[
            pltpu.VMEM((NCH, L), jnp.int32),
            pltpu.VMEM((VOCAB, HIDDEN), jnp.float32),
            pltpu.VMEM((PRE // WPB, HIDDEN), jnp.float32),
            pltpu.SemaphoreType.DMA,
            pltpu.SemaphoreType.DMA,
        ],
    )
    def k(ids_hbm, we_hbm, pe_hbm, out_hbm, idx_v, tab_v, pe_v, sem, sp):
        wid = lax.axis_index("s") * NC + lax.axis_index("c")
        b = wid // WPB
        q = wid % WPB
        s0 = q * ROWS_PER_W
        pltpu.sync_copy(ids_hbm.at[wid], idx_v)
        pltpu.sync_copy(we_hbm, tab_v)
        PQ = PRE // WPB
        pltpu.sync_copy(pe_hbm.at[pl.ds(q * PQ, PQ)], pe_v)
        pltpu.make_async_copy(
            pe_v, out_hbm.at[b, pl.ds(q * PQ, PQ)], sp
        ).start()

        def body(c, carry):
            vec = idx_v[c]
            row0 = PRE + s0 + c * L
            for j in range(L):
                sid = vec[j]
                pltpu.make_async_copy(
                    tab_v.at[pl.ds(sid, 1)],
                    out_hbm.at[b, pl.ds(row0 + j, 1)],
                    sem,
                ).start()
            return carry

        lax.fori_loop(0, NCH, body, 0)
        pltpu.make_async_copy(
            out_hbm.at[b, pl.ds(PRE + s0, ROWS_PER_W)],
            out_hbm.at[b, pl.ds(PRE + s0, ROWS_PER_W)],
            sem,
        ).wait()
        pltpu.make_async_copy(
            pe_v, out_hbm.at[b, pl.ds(q * (PRE // WPB), PRE // WPB)], sp
        ).wait()

    return k(ids3, word_embeddings, prompt_embeddings)


@jax.jit
def kernel(input_ids, word_embeddings, prompt_embeddings):
    ids3 = input_ids.astype(jnp.int32).reshape(NW, NCH, L)
    return _sc_embed(ids3, word_embeddings, prompt_embeddings)
